# Initial kernel scaffold; baseline (speedup 1.0000x reference)
import jax
import jax.numpy as jnp
from jax.experimental import pallas as pl


def kernel(input, mask):
    raise NotImplementedError("write your pallas kernel here")


# speedup vs baseline: 1.0719x; 1.0719x over previous
import functools

import jax
import jax.numpy as jnp
from jax import lax
from jax.experimental import pallas as pl
from jax.experimental.pallas import tpu as pltpu
from jax.experimental.pallas import tpu_sc as plsc

H = 64
W = 64
HW = H * W
C2 = 64
QT = 256
NQ = HW // QT

_NTILES = 32
_ROWS_PER_TILE = (2 * C2) // _NTILES


def _match_body(former_ref, latter_ref, flag_ref, ind_ref):
    lat = latter_ref[0]
    s = jnp.sum(lat * lat, axis=0, keepdims=True)
    latn = lat / (jnp.sqrt(s) + 1e-8)
    fmr = former_ref[0]
    sim = lax.dot_general(fmr.astype(jnp.bfloat16), latn.astype(jnp.bfloat16),
                          (((0,), (0,)), ((), ())),
                          preferred_element_type=jnp.float32)
    flag = flag_ref[0]
    sim = jnp.where(flag == 1, jnp.float32(-1e9), sim)
    half = HW // 2
    s1 = sim[:, :half]
    s2 = sim[:, half:]
    m1 = jnp.max(s1, axis=1, keepdims=True)
    m2 = jnp.max(s2, axis=1, keepdims=True)
    iota = lax.broadcasted_iota(jnp.int32, s1.shape, 1)
    i1 = jnp.min(jnp.where(s1 == m1, iota, jnp.int32(half)), axis=1)
    i2 = jnp.min(jnp.where(s2 == m2, iota, jnp.int32(half)), axis=1) + half
    b1 = m1.astype(jnp.bfloat16).astype(jnp.float32)
    ind = jnp.where(m2[:, 0] > b1[:, 0], i2, i1)
    ind_ref[0, 0, 0, :] = ind


@jax.jit
def _match(inp_r, flag3):
    out = pl.pallas_call(
        _match_body,
        grid=(2, NQ),
        in_specs=[
            pl.BlockSpec((1, C2, QT), lambda b, q: (b, 0, q)),
            pl.BlockSpec((1, C2, HW), lambda b, q: (b, 1, 0)),
            pl.BlockSpec((1, 1, HW), lambda b, q: (b, 0, 0)),
        ],
        out_specs=pl.BlockSpec((1, 1, 1, QT), lambda b, q: (b, q, 0, 0)),
        out_shape=jax.ShapeDtypeStruct((2, NQ, 1, QT), jnp.int32),
    )(inp_r, inp_r, flag3)
    return out.reshape(2, HW)


def _build_sc_shift():
    mesh = plsc.VectorSubcoreMesh(core_axis_name="c", subcore_axis_name="s")

    @functools.partial(
        pl.kernel,
        mesh=mesh,
        compiler_params=pltpu.CompilerParams(
            use_tc_tiling_on_sc=False, needs_layout_passes=False),
        out_type=jax.ShapeDtypeStruct((2, C2, HW), jnp.float32),
        scratch_types=[
            pltpu.VMEM((HW,), jnp.int32),
            pltpu.VMEM((HW,), jnp.int32),
            pltpu.VMEM((_ROWS_PER_TILE, HW), jnp.float32),
            pltpu.VMEM((_ROWS_PER_TILE, HW), jnp.float32),
        ],
    )
    def sc_shift(inp_hbm, ind_hbm, flag_hbm, out_hbm, idx_v, flg_v, lat_v, out_v):
        cid = lax.axis_index("c")
        sid = lax.axis_index("s")
        wid = sid * 2 + cid
        b = wid // 16
        c0 = (wid % 16) * _ROWS_PER_TILE
        pltpu.sync_copy(ind_hbm.at[b], idx_v)
        pltpu.sync_copy(flag_hbm.at[b], flg_v)
        pltpu.sync_copy(inp_hbm.at[b, pl.ds(C2 + c0, _ROWS_PER_TILE)], lat_v)

        def body(i, carry):
            off = i * 16
            idx = idx_v[pl.ds(off, 16)]
            keep = flg_v[pl.ds(off, 16)] == 1
            for c in range(_ROWS_PER_TILE):
                row = jnp.full((16,), c, jnp.int32)
                vals = plsc.load_gather(lat_v, [row, idx])
                out_v[c, pl.ds(off, 16)] = jnp.where(keep, vals, jnp.float32(0.0))
            return carry

        lax.fori_loop(0, HW // 16, body, 0)
        pltpu.sync_copy(out_v, out_hbm.at[b, pl.ds(c0, _ROWS_PER_TILE)])

    return sc_shift


_sc_shift_cache = []


def _sc_shift(*args):
    if not _sc_shift_cache:
        _sc_shift_cache.append(_build_sc_shift())
    return _sc_shift_cache[0](*args)


def kernel(input, mask):
    bz, c, h, w = input.shape
    c2 = c // 2
    inp_r = input.reshape(bz, c, h * w)
    flag = mask.reshape(bz, h * w)
    ind = _match(inp_r, flag.reshape(bz, 1, h * w))
    shift_t = _sc_shift(inp_r, ind, flag)
    return jnp.concatenate([input, shift_t.reshape(bz, C2, h, w)], axis=1)
